# baseline (speedup 1.0000x reference)
import jax
import jax.numpy as jnp
from jax.experimental import pallas as pl

N = 50000
E = 800000
IN_FEAT = 118
C = 64
H = 8
D = 8
L = 4
B = 256


def _layer(h, src, dst, Ws, Wd, a, b):
    xl = (h @ Ws).reshape(-1, H, D)
    xr = (h @ Wd).reshape(-1, H, D)
    m = xl[src] + xr[dst]
    e = jnp.sum(jax.nn.leaky_relu(m, 0.2) * a[None, :, :], axis=-1)
    emax = jax.ops.segment_max(e, dst, num_segments=N)
    emax = jnp.where(jnp.isfinite(emax), emax, 0.0)
    ee = jnp.exp(e - emax[dst])
    denom = jax.ops.segment_sum(ee, dst, num_segments=N)
    alpha = ee / (denom[dst] + 1e-16)
    msg = xl[src] * alpha[..., None]
    out = jax.ops.segment_sum(msg, dst, num_segments=N).reshape(-1, C) + b
    return out


def kernel(x, edge_index, batch, W_dense, b_dense, W_src, W_dst, att, b_conv, W_head, b_head):
    h = x @ W_dense + b_dense
    loop = jnp.arange(N, dtype=edge_index.dtype)
    src = jnp.concatenate([edge_index[0], loop])
    dst = jnp.concatenate([edge_index[1], loop])
    for l in range(L):
        h = _layer(h, src, dst, W_src[l], W_dst[l], att[l], b_conv[l])
    sums = jax.ops.segment_sum(h, batch, num_segments=B)
    counts = jax.ops.segment_sum(jnp.ones((N,), dtype=h.dtype), batch, num_segments=B)
    pooled = sums / jnp.clip(counts, 1.0)[:, None]
    out = pooled @ W_head + b_head
    return jnp.sinh(out)


# speedup vs baseline: 38.4187x; 38.4187x over previous
import functools

import jax
import jax.numpy as jnp
from jax import lax
from jax.experimental import pallas as pl
from jax.experimental.pallas import tpu as pltpu
from jax.experimental.pallas import tpu_sc as plsc

N = 50000
E = 800000
IN_FEAT = 118
C = 64
H = 8
D = 8
L = 4
B = 256

NC = 2
NS = 16
LANES = 16

CHUNK = 128
EPT_CHUNKS = 392
EPT = CHUNK * EPT_CHUNKS
EPAD = EPT * NS
NPS = 3136
NPAD = NPS * NS
NPS4 = NPS // 4
NPAD4 = NPAD // 4

BLK = 2000
GRID = N // BLK

f32 = jnp.float32
i32 = jnp.int32


def _dense_body(x_ref, w_ref, b_ref, o_ref):
    o_ref[...] = (
        jnp.dot(x_ref[...], w_ref[...], preferred_element_type=f32) + b_ref[...]
    )


def _run_dense(x, w, b):
    return pl.pallas_call(
        _dense_body,
        grid=(GRID,),
        in_specs=[
            pl.BlockSpec((BLK, IN_FEAT), lambda i: (i, 0)),
            pl.BlockSpec((IN_FEAT, C), lambda i: (0, 0)),
            pl.BlockSpec((1, C), lambda i: (0, 0)),
        ],
        out_specs=pl.BlockSpec((BLK, C), lambda i: (i, 0)),
        out_shape=jax.ShapeDtypeStruct((N, C), f32),
    )(x, w, b)


def _rep8(a4):
    return jnp.broadcast_to(a4[:, :, None], (BLK, 4, 8)).reshape(BLK, 32)


def _norm_h(num_ref, dp_ref, ees_ref, xlp_ref, bconv_ref):
    ees = ees_ref[...]
    halves = []
    for c in range(NC):
        den = dp_ref[c] + ees[:, 4 * c:4 * c + 4]
        inv = 1.0 / (den + 1e-16)
        numer = num_ref[c] + _rep8(ees[:, 4 * c:4 * c + 4]) * xlp_ref[c]
        halves.append(numer * _rep8(inv))
    return jnp.concatenate(halves, axis=1) + bconv_ref[...]


def _stage_a_common(hh, ws_ref, wd_ref, att_ref, xlp_ref, xrp_ref, ees_ref):
    xl = jnp.dot(hh, ws_ref[...], preferred_element_type=f32)
    xr = jnp.dot(hh, wd_ref[...], preferred_element_type=f32)
    m = xl + xr
    lk = jnp.maximum(m, 0.2 * m)
    es = jnp.sum((lk * att_ref[...]).reshape(BLK, H, D), axis=-1)
    ees_ref[...] = jnp.exp(es)
    xlp_ref[0, :, :] = xl[:, :32]
    xlp_ref[1, :, :] = xl[:, 32:]
    xrp_ref[0, :, :] = xr[:, :32]
    xrp_ref[1, :, :] = xr[:, 32:]


def _stage_a_h_body(h_ref, ws_ref, wd_ref, att_ref, xlp_ref, xrp_ref, ees_ref):
    _stage_a_common(h_ref[...], ws_ref, wd_ref, att_ref, xlp_ref, xrp_ref, ees_ref)


def _stage_a_parts_body(num_ref, dp_ref, ees_ref, xlpp_ref, bconv_ref,
                        ws_ref, wd_ref, att_ref,
                        xlp_ref, xrp_ref, ees_out_ref):
    hh = _norm_h(num_ref, dp_ref, ees_ref, xlpp_ref, bconv_ref)
    _stage_a_common(hh, ws_ref, wd_ref, att_ref, xlp_ref, xrp_ref, ees_out_ref)


_A_OUT = (
    jax.ShapeDtypeStruct((NC, N, 32), f32),
    jax.ShapeDtypeStruct((NC, N, 32), f32),
    jax.ShapeDtypeStruct((N, H), f32),
)
_A_OUT_SPECS = (
    pl.BlockSpec((NC, BLK, 32), lambda i: (0, i, 0)),
    pl.BlockSpec((NC, BLK, 32), lambda i: (0, i, 0)),
    pl.BlockSpec((BLK, H), lambda i: (i, 0)),
)


def _run_stage_a_h(h, ws, wd, attf):
    return pl.pallas_call(
        _stage_a_h_body,
        grid=(GRID,),
        in_specs=[
            pl.BlockSpec((BLK, C), lambda i: (i, 0)),
            pl.BlockSpec((C, C), lambda i: (0, 0)),
            pl.BlockSpec((C, C), lambda i: (0, 0)),
            pl.BlockSpec((1, C), lambda i: (0, 0)),
        ],
        out_specs=_A_OUT_SPECS,
        out_shape=_A_OUT,
    )(h, ws, wd, attf)


def _run_stage_a_parts(num, dpart, eeself, xlp_prev, bconv, ws, wd, attf):
    return pl.pallas_call(
        _stage_a_parts_body,
        grid=(GRID,),
        in_specs=[
            pl.BlockSpec((NC, BLK, 32), lambda i: (0, i, 0)),
            pl.BlockSpec((NC, BLK, 4), lambda i: (0, i, 0)),
            pl.BlockSpec((BLK, H), lambda i: (i, 0)),
            pl.BlockSpec((NC, BLK, 32), lambda i: (0, i, 0)),
            pl.BlockSpec((1, C), lambda i: (0, 0)),
            pl.BlockSpec((C, C), lambda i: (0, 0)),
            pl.BlockSpec((C, C), lambda i: (0, 0)),
            pl.BlockSpec((1, C), lambda i: (0, 0)),
        ],
        out_specs=_A_OUT_SPECS,
        out_shape=_A_OUT,
    )(num, dpart, eeself, xlp_prev, bconv, ws, wd, attf)


def _pool_body(num_ref, dp_ref, ees_ref, xlpp_ref, bconv_ref, batch_ref,
               wh_ref, bh_ref, out_ref, psum, cnt):
    i = pl.program_id(0)

    @pl.when(i == 0)
    def _init():
        psum[...] = jnp.zeros_like(psum)
        cnt[...] = jnp.zeros_like(cnt)

    hh = _norm_h(num_ref, dp_ref, ees_ref, xlpp_ref, bconv_ref)
    bids = batch_ref[0, 0, :]
    rows = lax.broadcasted_iota(i32, (B, BLK), 0)
    oh = (rows == bids[None, :]).astype(f32)
    psum[...] += jnp.dot(oh, hh, preferred_element_type=f32)
    cnt[...] += jnp.sum(oh, axis=1, keepdims=True)

    @pl.when(i == GRID - 1)
    def _fin():
        pooled = psum[...] / jnp.maximum(cnt[...], 1.0)
        r = jnp.sum(pooled * wh_ref[...], axis=1, keepdims=True) + bh_ref[...]
        out_ref[...] = 0.5 * (jnp.exp(r) - jnp.exp(-r))


def _run_pool(num, dpart, eeself, xlp_prev, bconv, batch3, whf, bh):
    return pl.pallas_call(
        _pool_body,
        grid=(GRID,),
        in_specs=[
            pl.BlockSpec((NC, BLK, 32), lambda i: (0, i, 0)),
            pl.BlockSpec((NC, BLK, 4), lambda i: (0, i, 0)),
            pl.BlockSpec((BLK, H), lambda i: (i, 0)),
            pl.BlockSpec((NC, BLK, 32), lambda i: (0, i, 0)),
            pl.BlockSpec((1, C), lambda i: (0, 0)),
            pl.BlockSpec((1, 1, BLK), lambda i: (i, 0, 0)),
            pl.BlockSpec((1, C), lambda i: (0, 0)),
            pl.BlockSpec((1, 1), lambda i: (0, 0)),
        ],
        out_specs=pl.BlockSpec((B, 1), lambda i: (0, 0)),
        out_shape=jax.ShapeDtypeStruct((B, 1), f32),
        scratch_shapes=[
            pltpu.VMEM((B, C), f32),
            pltpu.VMEM((B, 1), f32),
        ],
    )(num, dpart, eeself, xlp_prev, bconv, batch3, whf, bh)


def _edge_body(src_h, dst_h, xlp_h, xrp_h, att_h, z16_h, z32_h,
               num_h, dpart_h,
               sidxA, didxA, didx, didx4, bl, br, eebuf, msgbuf, attv, dsh, nsh,
               sem1, sem2):
    c = lax.axis_index("c")
    s = lax.axis_index("s")

    pltpu.sync_copy(z16_h.at[pl.ds(s * NPS4, NPS4)], dsh.at[pl.ds(s * NPS4, NPS4)])
    pltpu.sync_copy(z32_h.at[pl.ds(s * NPS, NPS)], nsh.at[pl.ds(s * NPS, NPS)])
    pltpu.sync_copy(att_h, attv)
    plsc.subcore_barrier()

    iota = lax.broadcasted_iota(i32, (LANES,), 0)
    half = c * N
    base_e = s * EPT

    def chunk_body(ci, carry):
        off = base_e + ci * CHUNK
        pltpu.sync_copy(src_h.at[pl.ds(off, CHUNK)], sidxA)
        pltpu.sync_copy(dst_h.at[pl.ds(off, CHUNK)], didx)

        def addn(i, c2):
            sl = pl.ds(i * LANES, LANES)
            sidxA[sl] = sidxA[sl] + half
            didxA[sl] = didx[sl] + half
            didx4[sl] = lax.shift_right_logical(didx[sl], 2)
            return c2

        lax.fori_loop(0, CHUNK // LANES, addn, 0)

        cp1 = pltpu.async_copy(xlp_h.at[sidxA], bl, sem1)
        cp2 = pltpu.async_copy(xrp_h.at[didxA], br, sem2)
        cp1.wait()
        cp2.wait()

        def group_body(g, c2):
            ids = g * LANES + iota
            gmask = (off + ids) < E
            dmod = jnp.bitwise_and(didx[pl.ds(g * LANES, LANES)], 3)
            ees = []
            for hh in range(4):
                xvs = []
                acc = jnp.zeros((LANES,), f32)
                for d in range(D):
                    col = hh * D + d
                    colv = jnp.full((LANES,), col, i32)
                    xv = plsc.load_gather(bl, [ids, colv])
                    rv = plsc.load_gather(br, [ids, colv])
                    av = plsc.load_gather(attv, [colv + c * 32])
                    m = xv + rv
                    acc = acc + jnp.maximum(m, 0.2 * m) * av
                    xvs.append(xv)
                eev = jnp.exp(acc)
                eev = jnp.where(gmask, eev, 0.0)
                ees.append(eev)
                for d in range(D):
                    col = hh * D + d
                    colv = jnp.full((LANES,), col, i32)
                    plsc.store_scatter(msgbuf, [ids, colv], xvs[d] * eev)
            for q in range(4):
                qm = dmod == q
                for hh in range(4):
                    colv = jnp.full((LANES,), q * 4 + hh, i32)
                    val = jnp.where(qm, ees[hh], 0.0)
                    plsc.store_scatter(eebuf, [ids, colv], val)
            return c2

        lax.fori_loop(0, CHUNK // LANES, group_body, 0)

        pltpu.sync_copy(eebuf, dsh.at[didx4], add=True)
        pltpu.sync_copy(msgbuf, nsh.at[didx], add=True)
        return carry

    lax.fori_loop(0, EPT_CHUNKS, chunk_body, 0)
    plsc.subcore_barrier()
    pltpu.sync_copy(dsh.at[pl.ds(s * NPS4, NPS4)],
                    dpart_h.at[c, pl.ds(s * NPS4, NPS4)])
    pltpu.sync_copy(nsh.at[pl.ds(s * NPS, NPS)],
                    num_h.at[c, pl.ds(s * NPS, NPS)])


@functools.cache
def _build_edge_pass():
    return functools.partial(
        pl.kernel,
        compiler_params=pltpu.CompilerParams(
            use_tc_tiling_on_sc=False, needs_layout_passes=False),
        out_type=(
            jax.ShapeDtypeStruct((NC, NPAD, 32), f32),
            jax.ShapeDtypeStruct((NC, NPAD4, 16), f32),
        ),
        mesh=plsc.VectorSubcoreMesh(
            core_axis_name="c", subcore_axis_name="s",
            num_cores=NC, num_subcores=NS),
        scratch_types=(
            pltpu.VMEM((CHUNK,), i32),
            pltpu.VMEM((CHUNK,), i32),
            pltpu.VMEM((CHUNK,), i32),
            pltpu.VMEM((CHUNK,), i32),
            pltpu.VMEM((CHUNK, 32), f32),
            pltpu.VMEM((CHUNK, 32), f32),
            pltpu.VMEM((CHUNK, 16), f32),
            pltpu.VMEM((CHUNK, 32), f32),
            pltpu.VMEM((C,), f32),
            pltpu.VMEM_SHARED((NPAD4, 16), f32),
            pltpu.VMEM_SHARED((NPAD, 32), f32),
            pltpu.SemaphoreType.DMA,
            pltpu.SemaphoreType.DMA,
        ),
    )(_edge_body)


def _edge_pass(srcp, dstp, xlp2, xrp2, attf, z16, z32):
    num, dpart4 = _build_edge_pass()(srcp, dstp, xlp2, xrp2, attf, z16, z32)
    return num, dpart4.reshape(NC, NPAD, 4)


def kernel(x, edge_index, batch, W_dense, b_dense, W_src, W_dst, att, b_conv,
           W_head, b_head):
    srcp = jnp.pad(edge_index[0], (0, EPAD - E))
    dstp = jnp.pad(edge_index[1], (0, EPAD - E))
    z16 = jnp.zeros((NPAD4, 16), f32)
    z32 = jnp.zeros((NPAD, 32), f32)

    h = _run_dense(x, W_dense, b_dense.reshape(1, C))
    num = dpart = eeself = xlp = None
    for l in range(L):
        attf = att[l].reshape(1, C)
        if l == 0:
            xlp, xrp, eeself = _run_stage_a_h(h, W_src[l], W_dst[l], attf)
        else:
            xlp, xrp, eeself = _run_stage_a_parts(
                num, dpart, eeself, xlp, b_conv[l - 1].reshape(1, C),
                W_src[l], W_dst[l], attf)
        num, dpart = _edge_pass(
            srcp, dstp, xlp.reshape(NC * N, 32), xrp.reshape(NC * N, 32),
            att[l].reshape(C), z16, z32)

    out = _run_pool(num, dpart, eeself, xlp, b_conv[L - 1].reshape(1, C),
                    batch.reshape(GRID, 1, BLK), W_head.reshape(1, C),
                    b_head.reshape(1, 1))
    return out


# speedup vs baseline: 47.1357x; 1.2269x over previous
import functools

import jax
import jax.numpy as jnp
from jax import lax
from jax.experimental import pallas as pl
from jax.experimental.pallas import tpu as pltpu
from jax.experimental.pallas import tpu_sc as plsc

N = 50000
E = 800000
IN_FEAT = 118
C = 64
H = 8
D = 8
L = 4
B = 256

NC = 2
NS = 16
LANES = 16

CHUNK = 96
EPT_CHUNKS = 522
EPT = CHUNK * EPT_CHUNKS
EPAD = EPT * NS
GPC = CHUNK // LANES
NPS = 3136
NPAD = NPS * NS
NPS4 = NPS // 4
NPAD4 = NPAD // 4

BLK = 2000
GRID = N // BLK

f32 = jnp.float32
i32 = jnp.int32


def _dense_body(x_ref, w_ref, b_ref, o_ref):
    o_ref[...] = (
        jnp.dot(x_ref[...], w_ref[...], preferred_element_type=f32) + b_ref[...]
    )


def _run_dense(x, w, b):
    return pl.pallas_call(
        _dense_body,
        grid=(GRID,),
        in_specs=[
            pl.BlockSpec((BLK, IN_FEAT), lambda i: (i, 0)),
            pl.BlockSpec((IN_FEAT, C), lambda i: (0, 0)),
            pl.BlockSpec((1, C), lambda i: (0, 0)),
        ],
        out_specs=pl.BlockSpec((BLK, C), lambda i: (i, 0)),
        out_shape=jax.ShapeDtypeStruct((N, C), f32),
    )(x, w, b)


def _rep8(a4):
    return jnp.broadcast_to(a4[:, :, None], (BLK, 4, 8)).reshape(BLK, 32)


def _norm_h(num_ref, dp_ref, ees_ref, xlp_ref, bconv_ref):
    ees = ees_ref[...]
    halves = []
    for c in range(NC):
        den = dp_ref[c] + ees[:, 4 * c:4 * c + 4]
        inv = 1.0 / (den + 1e-16)
        numer = num_ref[c] + _rep8(ees[:, 4 * c:4 * c + 4]) * xlp_ref[c]
        halves.append(numer * _rep8(inv))
    return jnp.concatenate(halves, axis=1) + bconv_ref[...]


def _stage_a_common(hh, ws_ref, wd_ref, att_ref, xlp_ref, xrp_ref, ees_ref):
    xl = jnp.dot(hh, ws_ref[...], preferred_element_type=f32)
    xr = jnp.dot(hh, wd_ref[...], preferred_element_type=f32)
    m = xl + xr
    lk = jnp.maximum(m, 0.2 * m)
    es = jnp.sum((lk * att_ref[...]).reshape(BLK, H, D), axis=-1)
    ees_ref[...] = jnp.exp(es)
    xlp_ref[0, :, :] = xl[:, :32]
    xlp_ref[1, :, :] = xl[:, 32:]
    xrp_ref[0, :, :] = xr[:, :32]
    xrp_ref[1, :, :] = xr[:, 32:]


def _stage_a_h_body(h_ref, ws_ref, wd_ref, att_ref, xlp_ref, xrp_ref, ees_ref):
    _stage_a_common(h_ref[...], ws_ref, wd_ref, att_ref, xlp_ref, xrp_ref, ees_ref)


def _stage_a_parts_body(num_ref, dp_ref, ees_ref, xlpp_ref, bconv_ref,
                        ws_ref, wd_ref, att_ref,
                        xlp_ref, xrp_ref, ees_out_ref):
    hh = _norm_h(num_ref, dp_ref, ees_ref, xlpp_ref, bconv_ref)
    _stage_a_common(hh, ws_ref, wd_ref, att_ref, xlp_ref, xrp_ref, ees_out_ref)


_A_OUT = (
    jax.ShapeDtypeStruct((NC, N, 32), f32),
    jax.ShapeDtypeStruct((NC, N, 32), f32),
    jax.ShapeDtypeStruct((N, H), f32),
)
_A_OUT_SPECS = (
    pl.BlockSpec((NC, BLK, 32), lambda i: (0, i, 0)),
    pl.BlockSpec((NC, BLK, 32), lambda i: (0, i, 0)),
    pl.BlockSpec((BLK, H), lambda i: (i, 0)),
)


def _run_stage_a_h(h, ws, wd, attf):
    return pl.pallas_call(
        _stage_a_h_body,
        grid=(GRID,),
        in_specs=[
            pl.BlockSpec((BLK, C), lambda i: (i, 0)),
            pl.BlockSpec((C, C), lambda i: (0, 0)),
            pl.BlockSpec((C, C), lambda i: (0, 0)),
            pl.BlockSpec((1, C), lambda i: (0, 0)),
        ],
        out_specs=_A_OUT_SPECS,
        out_shape=_A_OUT,
    )(h, ws, wd, attf)


def _run_stage_a_parts(num, dpart, eeself, xlp_prev, bconv, ws, wd, attf):
    return pl.pallas_call(
        _stage_a_parts_body,
        grid=(GRID,),
        in_specs=[
            pl.BlockSpec((NC, BLK, 32), lambda i: (0, i, 0)),
            pl.BlockSpec((NC, BLK, 4), lambda i: (0, i, 0)),
            pl.BlockSpec((BLK, H), lambda i: (i, 0)),
            pl.BlockSpec((NC, BLK, 32), lambda i: (0, i, 0)),
            pl.BlockSpec((1, C), lambda i: (0, 0)),
            pl.BlockSpec((C, C), lambda i: (0, 0)),
            pl.BlockSpec((C, C), lambda i: (0, 0)),
            pl.BlockSpec((1, C), lambda i: (0, 0)),
        ],
        out_specs=_A_OUT_SPECS,
        out_shape=_A_OUT,
    )(num, dpart, eeself, xlp_prev, bconv, ws, wd, attf)


def _pool_body(num_ref, dp_ref, ees_ref, xlpp_ref, bconv_ref, batch_ref,
               wh_ref, bh_ref, out_ref, psum, cnt):
    i = pl.program_id(0)

    @pl.when(i == 0)
    def _init():
        psum[...] = jnp.zeros_like(psum)
        cnt[...] = jnp.zeros_like(cnt)

    hh = _norm_h(num_ref, dp_ref, ees_ref, xlpp_ref, bconv_ref)
    bids = batch_ref[0, 0, :]
    rows = lax.broadcasted_iota(i32, (B, BLK), 0)
    oh = (rows == bids[None, :]).astype(f32)
    psum[...] += jnp.dot(oh, hh, preferred_element_type=f32)
    cnt[...] += jnp.sum(oh, axis=1, keepdims=True)

    @pl.when(i == GRID - 1)
    def _fin():
        pooled = psum[...] / jnp.maximum(cnt[...], 1.0)
        r = jnp.sum(pooled * wh_ref[...], axis=1, keepdims=True) + bh_ref[...]
        out_ref[...] = 0.5 * (jnp.exp(r) - jnp.exp(-r))


def _run_pool(num, dpart, eeself, xlp_prev, bconv, batch3, whf, bh):
    return pl.pallas_call(
        _pool_body,
        grid=(GRID,),
        in_specs=[
            pl.BlockSpec((NC, BLK, 32), lambda i: (0, i, 0)),
            pl.BlockSpec((NC, BLK, 4), lambda i: (0, i, 0)),
            pl.BlockSpec((BLK, H), lambda i: (i, 0)),
            pl.BlockSpec((NC, BLK, 32), lambda i: (0, i, 0)),
            pl.BlockSpec((1, C), lambda i: (0, 0)),
            pl.BlockSpec((1, 1, BLK), lambda i: (i, 0, 0)),
            pl.BlockSpec((1, C), lambda i: (0, 0)),
            pl.BlockSpec((1, 1), lambda i: (0, 0)),
        ],
        out_specs=pl.BlockSpec((B, 1), lambda i: (0, 0)),
        out_shape=jax.ShapeDtypeStruct((B, 1), f32),
        scratch_shapes=[
            pltpu.VMEM((B, C), f32),
            pltpu.VMEM((B, 1), f32),
        ],
    )(num, dpart, eeself, xlp_prev, bconv, batch3, whf, bh)


def _edge_body(idx_h, xlp_h, xrp_h, att_h, z16_h, z32_h,
               num_h, dpart_h,
               idxb0, idxb1, idxb2, bl0, bl1, br0, br1, eebuf, attv, dsh, nsh,
               semi0, semi1, semi2, sbl0, sbl1, sbr0, sbr1, semsn, semse):
    c = lax.axis_index("c")
    s = lax.axis_index("s")
    idxbs = (idxb0, idxb1, idxb2)
    semis = (semi0, semi1, semi2)
    bls = (bl0, bl1)
    brs = (br0, br1)
    sbls = (sbl0, sbl1)
    sbrs = (sbr0, sbr1)

    pltpu.sync_copy(z16_h.at[pl.ds(s * NPS4, NPS4)], dsh.at[pl.ds(s * NPS4, NPS4)])
    pltpu.sync_copy(z32_h.at[pl.ds(s * NPS, NPS)], nsh.at[pl.ds(s * NPS, NPS)])
    pltpu.sync_copy(att_h, attv)
    plsc.subcore_barrier()

    iota = lax.broadcasted_iota(i32, (LANES,), 0)
    base_e = s * EPT
    NCH = EPT_CHUNKS

    def issue_idx(ci, slot):
        return pltpu.async_copy(
            idx_h.at[pl.ds(c * 4, 4), pl.ds(base_e + ci * CHUNK, CHUNK)],
            idxbs[slot], semis[slot])

    def wait_idx(slot):
        pltpu.make_async_copy(
            idx_h.at[pl.ds(0, 4), pl.ds(0, CHUNK)], idxbs[slot],
            semis[slot]).wait()

    def issue_gathers(b, slot):
        pltpu.async_copy(xlp_h.at[idxbs[slot].at[0]], bls[b], sbls[b])
        pltpu.async_copy(xrp_h.at[idxbs[slot].at[1]], brs[b], sbrs[b])

    def wait_gathers(b, slot):
        pltpu.make_async_copy(xlp_h.at[idxbs[slot].at[0]], bls[b], sbls[b]).wait()
        pltpu.make_async_copy(xrp_h.at[idxbs[slot].at[1]], brs[b], sbrs[b]).wait()

    def do_scatters(b, slot):
        pltpu.sync_copy(bls[b], nsh.at[idxbs[slot].at[2]], add=True)
        pltpu.sync_copy(eebuf, dsh.at[idxbs[slot].at[3]], add=True)

    def compute(ci, b, slot):
        idxb = idxbs[slot]
        bl = bls[b]
        br = brs[b]

        def group_body(g, c2):
            ids = g * LANES + iota
            gmask = (base_e + ci * CHUNK + ids) < E
            dmod = jnp.bitwise_and(idxb[2, pl.ds(g * LANES, LANES)], 3)
            ees = []
            xvss = []
            for hh in range(4):
                xvs = []
                acc = jnp.zeros((LANES,), f32)
                for d in range(D):
                    col = hh * D + d
                    colv = jnp.full((LANES,), col, i32)
                    xv = plsc.load_gather(bl, [ids, colv])
                    rv = plsc.load_gather(br, [ids, colv])
                    av = plsc.load_gather(attv, [colv + c * 32])
                    m = xv + rv
                    acc = acc + jnp.maximum(m, 0.2 * m) * av
                    xvs.append(xv)
                eev = jnp.exp(acc)
                eev = jnp.where(gmask, eev, 0.0)
                ees.append(eev)
                xvss.append(xvs)
            for hh in range(4):
                for d in range(D):
                    colv = jnp.full((LANES,), hh * D + d, i32)
                    plsc.store_scatter(bl, [ids, colv], xvss[hh][d] * ees[hh])
            for q in range(4):
                qm = dmod == q
                for hh in range(4):
                    colv = jnp.full((LANES,), q * 4 + hh, i32)
                    plsc.store_scatter(eebuf, [ids, colv],
                                       jnp.where(qm, ees[hh], 0.0))
            return c2

        lax.fori_loop(0, GPC, group_body, 0)

    issue_idx(0, 0)
    issue_idx(1, 1)
    wait_idx(0)
    issue_gathers(0, 0)

    def outer_body(ci6, carry):
        for k in range(6):
            ci = ci6 * 6 + k
            b = k % 2
            slot = k % 3
            wait_gathers(b, slot)

            @pl.when(ci + 1 < NCH)
            def _next_gather():
                wait_idx((slot + 1) % 3)
                issue_gathers(1 - b, (slot + 1) % 3)

            @pl.when(ci + 2 < NCH)
            def _next_idx():
                issue_idx(ci + 2, (slot + 2) % 3)

            compute(ci, b, slot)
            do_scatters(b, slot)
        return carry

    lax.fori_loop(0, NCH // 6, outer_body, 0)
    plsc.subcore_barrier()
    pltpu.sync_copy(dsh.at[pl.ds(s * NPS4, NPS4)],
                    dpart_h.at[c, pl.ds(s * NPS4, NPS4)])
    pltpu.sync_copy(nsh.at[pl.ds(s * NPS, NPS)],
                    num_h.at[c, pl.ds(s * NPS, NPS)])


@functools.cache
def _build_edge_pass():
    return functools.partial(
        pl.kernel,
        compiler_params=pltpu.CompilerParams(
            use_tc_tiling_on_sc=False, needs_layout_passes=False),
        out_type=(
            jax.ShapeDtypeStruct((NC, NPAD, 32), f32),
            jax.ShapeDtypeStruct((NC, NPAD4, 16), f32),
        ),
        mesh=plsc.VectorSubcoreMesh(
            core_axis_name="c", subcore_axis_name="s",
            num_cores=NC, num_subcores=NS),
        scratch_types=(
            pltpu.VMEM((4, CHUNK), i32),
            pltpu.VMEM((4, CHUNK), i32),
            pltpu.VMEM((4, CHUNK), i32),
            pltpu.VMEM((CHUNK, 32), f32),
            pltpu.VMEM((CHUNK, 32), f32),
            pltpu.VMEM((CHUNK, 32), f32),
            pltpu.VMEM((CHUNK, 32), f32),
            pltpu.VMEM((CHUNK, 16), f32),
            pltpu.VMEM((C,), f32),
            pltpu.VMEM_SHARED((NPAD4, 16), f32),
            pltpu.VMEM_SHARED((NPAD, 32), f32),
            pltpu.SemaphoreType.DMA,
            pltpu.SemaphoreType.DMA,
            pltpu.SemaphoreType.DMA,
            pltpu.SemaphoreType.DMA,
            pltpu.SemaphoreType.DMA,
            pltpu.SemaphoreType.DMA,
            pltpu.SemaphoreType.DMA,
            pltpu.SemaphoreType.DMA,
            pltpu.SemaphoreType.DMA,
        ),
    )(_edge_body)


def _edge_pass(idxT, xlp2, xrp2, attf, z16, z32):
    num, dpart4 = _build_edge_pass()(idxT, xlp2, xrp2, attf, z16, z32)
    return num, dpart4.reshape(NC, NPAD, 4)


def kernel(x, edge_index, batch, W_dense, b_dense, W_src, W_dst, att, b_conv,
           W_head, b_head):
    srcp = jnp.pad(edge_index[0], (0, EPAD - E))
    dstp = jnp.pad(edge_index[1], (0, EPAD - E))
    idxT = jnp.concatenate(
        [jnp.stack([srcp + c * N, dstp + c * N, dstp, dstp // 4])
         for c in range(NC)], axis=0)
    z16 = jnp.zeros((NPAD4, 16), f32)
    z32 = jnp.zeros((NPAD, 32), f32)

    h = _run_dense(x, W_dense, b_dense.reshape(1, C))
    num = dpart = eeself = xlp = None
    for l in range(L):
        attf = att[l].reshape(1, C)
        if l == 0:
            xlp, xrp, eeself = _run_stage_a_h(h, W_src[l], W_dst[l], attf)
        else:
            xlp, xrp, eeself = _run_stage_a_parts(
                num, dpart, eeself, xlp, b_conv[l - 1].reshape(1, C),
                W_src[l], W_dst[l], attf)
        num, dpart = _edge_pass(
            idxT, xlp.reshape(NC * N, 32), xrp.reshape(NC * N, 32),
            att[l].reshape(C), z16, z32)

    out = _run_pool(num, dpart, eeself, xlp, b_conv[L - 1].reshape(1, C),
                    batch.reshape(GRID, 1, BLK), W_head.reshape(1, C),
                    b_head.reshape(1, 1))
    return out
